# baseline (speedup 1.0000x reference)
import jax
import jax.numpy as jnp
from jax.experimental import pallas as pl
from jax.experimental.pallas import tpu as pltpu

S_ = 2048
DIM_ = 1024
DH_ = 64
H_ = 16
KVH_ = 4
GQ_ = H_ // KVH_
BLK_ = 64
NB_ = S_ // BLK_
NSEL_ = 8
NMEM_ = 1
HID_ = BLK_ * DH_
SCALE_ = DH_ ** -0.5
MNEG_ = float(-jnp.finfo(jnp.float32).max)
EPS_ = float(jnp.finfo(jnp.float32).eps)

_RA = 256
_HT = 512


def _qkv_body(x_ref, wt_ref, nw_ref, bg_ref, cosq_ref, sinq_ref, cosk_ref,
              sink_ref, q_ref, qr_ref, k_ref, kr_ref, v_ref, g_ref):
    x = x_ref[...]
    ms = jnp.mean(x * x, axis=1, keepdims=True)
    x = x * jax.lax.rsqrt(ms + EPS_) * nw_ref[...]
    y = jnp.dot(x, wt_ref[...], preferred_element_type=jnp.float32)
    q = y[:, : H_ * DH_]
    k = y[:, H_ * DH_: (H_ + KVH_) * DH_]
    v = y[:, (H_ + KVH_) * DH_: (H_ + 2 * KVH_) * DH_]
    graw = y[:, (H_ + 2 * KVH_) * DH_:]
    g_ref[...] = jax.nn.sigmoid(graw + bg_ref[...])
    q_ref[...] = q
    k_ref[...] = k
    v_ref[...] = v

    def rot(t, cos, sin):
        a = jnp.concatenate([t[:, -1:], t[:, :-1]], axis=1)
        b = jnp.concatenate([t[:, 1:], t[:, :1]], axis=1)
        lane = jax.lax.broadcasted_iota(jnp.int32, t.shape, 1)
        rh = jnp.where(lane % 2 == 0, -b, a)
        return t * cos + rh * sin

    qr_ref[...] = rot(q, cosq_ref[...], sinq_ref[...])
    kr_ref[...] = rot(k, cosk_ref[...], sink_ref[...])


def _comp_body(ak_ref, av_ref, kp_ref, vp_ref, w1k_ref, b1k_ref, w2k_ref,
               w1v_ref, b1v_ref, w2v_ref, bk2_ref, bv2_ref, ck_ref, cv_ref):
    t = pl.program_id(0)

    @pl.when(t == 0)
    def _():
        ck_ref[...] = jnp.broadcast_to(bk2_ref[...], ck_ref.shape)
        cv_ref[...] = jnp.broadcast_to(bv2_ref[...], cv_ref.shape)

    ak = ak_ref[...] + kp_ref[...]
    av = av_ref[...] + vp_ref[...]
    hk = jnp.maximum(
        jnp.dot(ak, w1k_ref[...], preferred_element_type=jnp.float32)
        + b1k_ref[...], 0.0)
    hv = jnp.maximum(
        jnp.dot(av, w1v_ref[...], preferred_element_type=jnp.float32)
        + b1v_ref[...], 0.0)
    ck_ref[...] += jnp.dot(hk, w2k_ref[...], preferred_element_type=jnp.float32)
    cv_ref[...] += jnp.dot(hv, w2v_ref[...], preferred_element_type=jnp.float32)


def _attn_body(qp_ref, qr_ref, kr_ref, v_ref, ck_ref, cv_ref, g0_ref, g1_ref,
               out_ref):
    qb = pl.program_id(1)
    qp = qp_ref[0, 0]
    csim = jnp.dot(qp, ck_ref[0].T, preferred_element_type=jnp.float32) * SCALE_
    col = jax.lax.broadcasted_iota(jnp.int32, csim.shape, 1)
    csim = jnp.where(col < qb + 1, csim, MNEG_)
    cmax = jnp.max(csim, axis=1, keepdims=True)
    cp = jnp.exp(csim - cmax)
    cattn = cp / jnp.sum(cp, axis=1, keepdims=True)
    comp = jnp.dot(cattn, cv_ref[0], preferred_element_type=jnp.float32)

    imp = jnp.mean(csim.reshape(GQ_, BLK_, NB_ + NMEM_), axis=0)[:, NMEM_:]
    m2 = jnp.maximum(jnp.max(imp, axis=1, keepdims=True), -1e3)
    e = jnp.exp(imp - m2)
    probs = e / (jnp.sum(e, axis=1, keepdims=True) + jnp.exp(-1e3 - m2))

    coli = jax.lax.broadcasted_iota(jnp.int32, probs.shape, 1)
    sel = jnp.zeros(probs.shape, jnp.float32)
    work = probs
    for _ in range(NSEL_):
        m = jnp.max(work, axis=1, keepdims=True)
        cand = work == m
        idxv = jnp.where(cand, coli, NB_ * 2)
        pick = coli == jnp.min(idxv, axis=1, keepdims=True)
        sel = jnp.where(pick & (m > 1e-10), 1.0, sel)
        work = jnp.where(pick, -1.0, work)

    qr = qr_ref[0, 0]
    piota = jax.lax.broadcasted_iota(jnp.int32, (BLK_, BLK_), 0)
    kiota = jax.lax.broadcasted_iota(jnp.int32, (BLK_, BLK_), 1)

    def step(j, carry):
        m_i, l_i, acc = carry
        kj = kr_ref[0, pl.ds(j * BLK_, BLK_), :]
        vj = v_ref[0, pl.ds(j * BLK_, BLK_), :]
        sim = jnp.dot(qr, kj.T, preferred_element_type=jnp.float32) * SCALE_
        selj = jnp.sum(sel * (coli == j).astype(jnp.float32), axis=1,
                       keepdims=True) > 0.0
        own = (j == qb) & (kiota <= piota)
        vis = selj | own
        sim = jnp.where(vis[None], sim.reshape(GQ_, BLK_, BLK_), MNEG_)
        sim = sim.reshape(GQ_ * BLK_, BLK_)
        m_new = jnp.maximum(m_i, jnp.max(sim, axis=1, keepdims=True))
        alpha = jnp.exp(m_i - m_new)
        p = jnp.exp(sim - m_new)
        l_new = l_i * alpha + jnp.sum(p, axis=1, keepdims=True)
        acc_new = acc * alpha + jnp.dot(p, vj, preferred_element_type=jnp.float32)
        return m_new, l_new, acc_new

    m0 = jnp.full((GQ_ * BLK_, 1), MNEG_, jnp.float32)
    l0 = jnp.zeros((GQ_ * BLK_, 1), jnp.float32)
    a0 = jnp.zeros((GQ_ * BLK_, DH_), jnp.float32)
    m_f, l_f, acc = jax.lax.fori_loop(0, qb + 1, step, (m0, l0, a0))
    fine = (acc / l_f).reshape(GQ_, BLK_, DH_)
    comp3 = comp.reshape(GQ_, BLK_, DH_)
    g0 = g0_ref[0, 0]
    g1 = g1_ref[0, 0]
    out_ref[0, 0] = g0[:, :, None] * comp3 + g1[:, :, None] * fine


def _proj_body(o_ref, w_ref, out_ref):
    out_ref[...] = jnp.dot(o_ref[...], w_ref[...],
                           preferred_element_type=jnp.float32)


def kernel(inp, norm_w, W_qkv, k_pos, v_pos, mem_kv, Wk1, bk1, Wk2, bk2,
           Wv1, bv1, Wv2, bv2, Wg, bg, Wo):
    x2 = inp.reshape(S_, DIM_)
    wt = jnp.concatenate([W_qkv, Wg], axis=0).T
    pos = jnp.arange(S_, dtype=jnp.float32)
    inv = 1.0 / (10000.0 ** (jnp.arange(0, DH_, 2, dtype=jnp.float32) / DH_))
    ang = pos[:, None] * inv[None, :]
    cos = jnp.repeat(jnp.cos(ang), 2, axis=-1)
    sin = jnp.repeat(jnp.sin(ang), 2, axis=-1)
    cosq = jnp.tile(cos, (1, H_))
    sinq = jnp.tile(sin, (1, H_))
    cosk = jnp.tile(cos, (1, KVH_))
    sink = jnp.tile(sin, (1, KVH_))

    na = S_ // _RA
    row_spec = lambda w: pl.BlockSpec((_RA, w), lambda i: (i, 0))
    full_spec = lambda a, b: pl.BlockSpec((a, b), lambda i: (0, 0))
    q, qr, k, kr, v, g = pl.pallas_call(
        _qkv_body,
        grid=(na,),
        in_specs=[row_spec(DIM_), full_spec(DIM_, (H_ + 2 * KVH_) * DH_ + 2 * H_),
                  full_spec(1, DIM_), full_spec(1, 2 * H_),
                  row_spec(H_ * DH_), row_spec(H_ * DH_),
                  row_spec(KVH_ * DH_), row_spec(KVH_ * DH_)],
        out_specs=[row_spec(H_ * DH_), row_spec(H_ * DH_),
                   row_spec(KVH_ * DH_), row_spec(KVH_ * DH_),
                   row_spec(KVH_ * DH_), row_spec(2 * H_)],
        out_shape=[jax.ShapeDtypeStruct((S_, H_ * DH_), jnp.float32),
                   jax.ShapeDtypeStruct((S_, H_ * DH_), jnp.float32),
                   jax.ShapeDtypeStruct((S_, KVH_ * DH_), jnp.float32),
                   jax.ShapeDtypeStruct((S_, KVH_ * DH_), jnp.float32),
                   jax.ShapeDtypeStruct((S_, KVH_ * DH_), jnp.float32),
                   jax.ShapeDtypeStruct((S_, 2 * H_), jnp.float32)],
    )(x2, wt, norm_w.reshape(1, DIM_), bg.reshape(1, 2 * H_),
      cosq, sinq, cosk, sink)

    ak = k.reshape(NB_, BLK_, KVH_, DH_).transpose(2, 0, 1, 3).reshape(
        KVH_ * NB_, HID_)
    av = v.reshape(NB_, BLK_, KVH_, DH_).transpose(2, 0, 1, 3).reshape(
        KVH_ * NB_, HID_)
    kp = jnp.tile(k_pos.reshape(KVH_, 1, HID_), (1, NB_, 1)).reshape(
        KVH_ * NB_, HID_)
    vp = jnp.tile(v_pos.reshape(KVH_, 1, HID_), (1, NB_, 1)).reshape(
        KVH_ * NB_, HID_)
    nt = HID_ // _HT
    ck_b, cv_b = pl.pallas_call(
        _comp_body,
        grid=(nt,),
        in_specs=[pl.BlockSpec((KVH_ * NB_, HID_), lambda t: (0, 0)),
                  pl.BlockSpec((KVH_ * NB_, HID_), lambda t: (0, 0)),
                  pl.BlockSpec((KVH_ * NB_, HID_), lambda t: (0, 0)),
                  pl.BlockSpec((KVH_ * NB_, HID_), lambda t: (0, 0)),
                  pl.BlockSpec((HID_, _HT), lambda t: (0, t)),
                  pl.BlockSpec((1, _HT), lambda t: (0, t)),
                  pl.BlockSpec((_HT, DH_), lambda t: (t, 0)),
                  pl.BlockSpec((HID_, _HT), lambda t: (0, t)),
                  pl.BlockSpec((1, _HT), lambda t: (0, t)),
                  pl.BlockSpec((_HT, DH_), lambda t: (t, 0)),
                  pl.BlockSpec((1, DH_), lambda t: (0, 0)),
                  pl.BlockSpec((1, DH_), lambda t: (0, 0))],
        out_specs=[pl.BlockSpec((KVH_ * NB_, DH_), lambda t: (0, 0)),
                   pl.BlockSpec((KVH_ * NB_, DH_), lambda t: (0, 0))],
        out_shape=[jax.ShapeDtypeStruct((KVH_ * NB_, DH_), jnp.float32),
                   jax.ShapeDtypeStruct((KVH_ * NB_, DH_), jnp.float32)],
    )(ak, av, kp, vp, Wk1.T, bk1.reshape(1, HID_), Wk2.T,
      Wv1.T, bv1.reshape(1, HID_), Wv2.T,
      bk2.reshape(1, DH_), bv2.reshape(1, DH_))

    ck = jnp.concatenate([mem_kv[0], ck_b.reshape(KVH_, NB_, DH_)], axis=1)
    cv = jnp.concatenate([mem_kv[1], cv_b.reshape(KVH_, NB_, DH_)], axis=1)

    def arr_q(t):
        return t.reshape(NB_, BLK_, KVH_, GQ_, DH_).transpose(
            2, 0, 3, 1, 4).reshape(KVH_, NB_, GQ_ * BLK_, DH_)

    qp_r = arr_q(q)
    qr_r = arr_q(qr)
    kr_r = kr.reshape(S_, KVH_, DH_).transpose(1, 0, 2)
    v_r = v.reshape(S_, KVH_, DH_).transpose(1, 0, 2)
    gg = g.reshape(S_, H_, 2)
    g0 = gg[..., 0].reshape(NB_, BLK_, KVH_, GQ_).transpose(2, 0, 3, 1)
    g1 = gg[..., 1].reshape(NB_, BLK_, KVH_, GQ_).transpose(2, 0, 3, 1)

    outc = pl.pallas_call(
        _attn_body,
        grid=(KVH_, NB_),
        in_specs=[
            pl.BlockSpec((1, 1, GQ_ * BLK_, DH_), lambda h, b: (h, b, 0, 0)),
            pl.BlockSpec((1, 1, GQ_ * BLK_, DH_), lambda h, b: (h, b, 0, 0)),
            pl.BlockSpec((1, S_, DH_), lambda h, b: (h, 0, 0)),
            pl.BlockSpec((1, S_, DH_), lambda h, b: (h, 0, 0)),
            pl.BlockSpec((1, NB_ + NMEM_, DH_), lambda h, b: (h, 0, 0)),
            pl.BlockSpec((1, NB_ + NMEM_, DH_), lambda h, b: (h, 0, 0)),
            pl.BlockSpec((1, 1, GQ_, BLK_), lambda h, b: (h, b, 0, 0)),
            pl.BlockSpec((1, 1, GQ_, BLK_), lambda h, b: (h, b, 0, 0)),
        ],
        out_specs=pl.BlockSpec((1, 1, GQ_, BLK_, DH_),
                               lambda h, b: (h, b, 0, 0, 0)),
        out_shape=jax.ShapeDtypeStruct((KVH_, NB_, GQ_, BLK_, DH_),
                                       jnp.float32),
    )(qp_r, qr_r, kr_r, v_r, ck, cv, g0, g1)

    o = outc.transpose(1, 3, 0, 2, 4).reshape(S_, H_ * DH_)
    out = pl.pallas_call(
        _proj_body,
        grid=(na,),
        in_specs=[row_spec(H_ * DH_), full_spec(H_ * DH_, DIM_)],
        out_specs=row_spec(DIM_),
        out_shape=jax.ShapeDtypeStruct((S_, DIM_), jnp.float32),
    )(o, Wo.T)
    return out.reshape(1, S_, DIM_)


# speedup vs baseline: 2.9085x; 2.9085x over previous
import jax
import jax.numpy as jnp
from jax.experimental import pallas as pl
from jax.experimental.pallas import tpu as pltpu

S_ = 2048
DIM_ = 1024
DH_ = 64
H_ = 16
KVH_ = 4
GQ_ = H_ // KVH_
BLK_ = 64
NB_ = S_ // BLK_
NSEL_ = 8
NMEM_ = 1
HID_ = BLK_ * DH_
SCALE_ = DH_ ** -0.5
MNEG_ = float(-jnp.finfo(jnp.float32).max)
EPS_ = float(jnp.finfo(jnp.float32).eps)

_RA = 256
_HT = 512
_GP = 256
_NG = S_ // _GP
_AUG = DH_ + NB_


def _rot(t, cos, sin):
    a = jnp.concatenate([t[:, -1:], t[:, :-1]], axis=1)
    b = jnp.concatenate([t[:, 1:], t[:, :1]], axis=1)
    lane = jax.lax.broadcasted_iota(jnp.int32, t.shape, 1)
    rh = jnp.where(lane % 2 == 0, -b, a)
    return t * cos + rh * sin


def _qkv_body(x_ref, wt_ref, nw_ref, bg_ref, q_ref, k_ref, v_ref, g_ref):
    x = x_ref[...]
    ms = jnp.mean(x * x, axis=1, keepdims=True)
    x = x * jax.lax.rsqrt(ms + EPS_) * nw_ref[...]
    y = jnp.dot(x, wt_ref[...], preferred_element_type=jnp.float32)
    q_ref[...] = y[:, :H_ * DH_]
    k_ref[...] = y[:, H_ * DH_:(H_ + KVH_) * DH_]
    v_ref[...] = y[:, (H_ + KVH_) * DH_:(H_ + 2 * KVH_) * DH_]
    g_ref[...] = jax.nn.sigmoid(y[:, (H_ + 2 * KVH_) * DH_:] + bg_ref[...])


def _comp_body(ak_ref, av_ref, kp_ref, vp_ref, w1k_ref, b1k_ref, w2k_ref,
               w1v_ref, b1v_ref, w2v_ref, bk2_ref, bv2_ref, ck_ref, cv_ref):
    t = pl.program_id(0)

    @pl.when(t == 0)
    def _():
        ck_ref[...] = jnp.broadcast_to(bk2_ref[...], ck_ref.shape)
        cv_ref[...] = jnp.broadcast_to(bv2_ref[...], cv_ref.shape)

    ak = (ak_ref[...].reshape(KVH_, NB_, HID_)
          + kp_ref[...][:, None, :]).reshape(KVH_ * NB_, HID_)
    av = (av_ref[...].reshape(KVH_, NB_, HID_)
          + vp_ref[...][:, None, :]).reshape(KVH_ * NB_, HID_)
    hk = jnp.maximum(
        jnp.dot(ak, w1k_ref[...], preferred_element_type=jnp.float32)
        + b1k_ref[...], 0.0)
    hv = jnp.maximum(
        jnp.dot(av, w1v_ref[...], preferred_element_type=jnp.float32)
        + b1v_ref[...], 0.0)
    ck_ref[...] += jnp.dot(hk, w2k_ref[...], preferred_element_type=jnp.float32)
    cv_ref[...] += jnp.dot(hv, w2v_ref[...], preferred_element_type=jnp.float32)


def _attn_body(q_ref, kv_ref, ck_ref, cv_ref, aux_ref, tri_ref,
               comp_ref, fine_ref, kr_ref):
    kr_ref[...] = jnp.concatenate(
        [_rot(kv_ref[0, :, :DH_], aux_ref[:, :DH_], aux_ref[:, DH_:2 * DH_]),
         aux_ref[:, 2 * DH_:2 * DH_ + NB_]], axis=1)
    ckt = ck_ref[0]
    cvt = cv_ref[0]
    for qg in range(_NG):
        lo = qg * _GP
        span = lo + _GP
        cosg = aux_ref[lo:span, :DH_]
        sing = aux_ref[lo:span, DH_:2 * DH_]
        csims = []
        for g in range(GQ_):
            qpg = q_ref[lo:span, g * DH_:(g + 1) * DH_]
            csims.append(jax.lax.dot_general(
                qpg, ckt, (((1,), (1,)), ((), ())),
                preferred_element_type=jnp.float32) * SCALE_)
        ri = jax.lax.broadcasted_iota(jnp.int32, csims[0].shape, 0)
        ci = jax.lax.broadcasted_iota(jnp.int32, csims[0].shape, 1)
        cvis = ci < (lo + ri) // BLK_ + 1
        imp = jnp.zeros(csims[0].shape, jnp.float32)
        for g in range(GQ_):
            cs = jnp.where(cvis, csims[g], MNEG_)
            imp = imp + cs
            cmax = jnp.max(cs, axis=1, keepdims=True)
            cp = jnp.exp(cs - cmax)
            cattn = cp / jnp.sum(cp, axis=1, keepdims=True)
            comp_ref[lo:span, g * DH_:(g + 1) * DH_] = jnp.dot(
                cattn, cvt, preferred_element_type=jnp.float32)

        imp = imp[:, NMEM_:] * (1.0 / GQ_)
        m2 = jnp.maximum(jnp.max(imp, axis=1, keepdims=True), -1e3)
        e = jnp.exp(imp - m2)
        probs = e / (jnp.sum(e, axis=1, keepdims=True) + jnp.exp(-1e3 - m2))

        work = probs.T
        rowi = jax.lax.broadcasted_iota(jnp.int32, work.shape, 0)
        sel_t = jnp.full(work.shape, -1e30, jnp.float32)
        for _ in range(NSEL_):
            m = jnp.max(work, axis=0, keepdims=True)
            cand = work == m
            idxv = jnp.where(cand, rowi, NB_ * 2)
            pick = rowi == jnp.min(idxv, axis=0, keepdims=True)
            sel_t = jnp.where(pick & (m > 1e-10), 0.0, sel_t)
            work = jnp.where(pick, -1.0, work)
        sel_pos = sel_t.T

        for g in range(GQ_):
            qrg = _rot(q_ref[lo:span, g * DH_:(g + 1) * DH_],
                       cosg, sing) * SCALE_
            aq = jnp.concatenate([qrg, sel_pos], axis=1)
            s_all = jax.lax.dot_general(
                aq, kr_ref[0:span, :], (((1,), (1,)), ((), ())),
                preferred_element_type=jnp.float32)
            ods = []
            for b in range(GQ_):
                qb_ = qrg[b * BLK_:(b + 1) * BLK_, :]
                kb_ = kr_ref[lo + b * BLK_:lo + (b + 1) * BLK_, :DH_]
                ods.append(jax.lax.dot_general(
                    qb_, kb_, (((1,), (1,)), ((), ())),
                    preferred_element_type=jnp.float32) + tri_ref[...])
            s_od = jnp.concatenate(ods, axis=0)
            mm = jnp.maximum(jnp.max(s_all, axis=1, keepdims=True),
                             jnp.max(s_od, axis=1, keepdims=True))
            p_all = jnp.exp(s_all - mm)
            p_od = jnp.exp(s_od - mm)
            l_f = (jnp.sum(p_all, axis=1, keepdims=True)
                   + jnp.sum(p_od, axis=1, keepdims=True))
            acc = jnp.dot(p_all, kv_ref[0, 0:span, DH_:],
                          preferred_element_type=jnp.float32)
            oacc = []
            for b in range(GQ_):
                oacc.append(jnp.dot(
                    p_od[b * BLK_:(b + 1) * BLK_, :],
                    kv_ref[0, lo + b * BLK_:lo + (b + 1) * BLK_, DH_:],
                    preferred_element_type=jnp.float32))
            acc = acc + jnp.concatenate(oacc, axis=0)
            fine_ref[lo:span, g * DH_:(g + 1) * DH_] = acc / l_f


def _proj_body(comp_ref, fine_ref, g_ref, e0_ref, e1_ref, w_ref, out_ref):
    g = g_ref[...]
    g0 = jnp.dot(g, e0_ref[...], preferred_element_type=jnp.float32)
    g1 = jnp.dot(g, e1_ref[...], preferred_element_type=jnp.float32)
    oh = g0 * comp_ref[...] + g1 * fine_ref[...]
    out_ref[...] = jnp.dot(oh, w_ref[...], preferred_element_type=jnp.float32)


def kernel(inp, norm_w, W_qkv, k_pos, v_pos, mem_kv, Wk1, bk1, Wk2, bk2,
           Wv1, bv1, Wv2, bv2, Wg, bg, Wo):
    x2 = inp.reshape(S_, DIM_)
    wt = jnp.concatenate([W_qkv, Wg], axis=0).T
    posf = jnp.arange(S_, dtype=jnp.float32)
    inv = 1.0 / (10000.0 ** (jnp.arange(0, DH_, 2, dtype=jnp.float32) / DH_))
    ang = posf[:, None] * inv[None, :]
    cos = jnp.repeat(jnp.cos(ang), 2, axis=-1)
    sin = jnp.repeat(jnp.sin(ang), 2, axis=-1)
    mt = jnp.repeat(jnp.eye(NB_, dtype=jnp.float32), BLK_, axis=1).T
    aux = jnp.concatenate(
        [cos, sin, mt, jnp.zeros((S_, 2 * DH_ - NB_), jnp.float32)], axis=1)
    ti = jnp.arange(BLK_)
    tri = jnp.where(ti[None, :] <= ti[:, None], 0.0, -1e30).astype(jnp.float32)

    na = S_ // _RA
    row_spec = lambda w: pl.BlockSpec((_RA, w), lambda i: (i, 0))
    q, k, v, g = pl.pallas_call(
        _qkv_body,
        grid=(na,),
        in_specs=[row_spec(DIM_),
                  pl.BlockSpec((DIM_, (H_ + 2 * KVH_) * DH_ + 2 * H_),
                               lambda i: (0, 0)),
                  pl.BlockSpec((1, DIM_), lambda i: (0, 0)),
                  pl.BlockSpec((1, 2 * H_), lambda i: (0, 0))],
        out_specs=[row_spec(H_ * DH_), row_spec(KVH_ * DH_),
                   row_spec(KVH_ * DH_), row_spec(2 * H_)],
        out_shape=[jax.ShapeDtypeStruct((S_, H_ * DH_), jnp.float32),
                   jax.ShapeDtypeStruct((S_, KVH_ * DH_), jnp.float32),
                   jax.ShapeDtypeStruct((S_, KVH_ * DH_), jnp.float32),
                   jax.ShapeDtypeStruct((S_, 2 * H_), jnp.float32)],
    )(x2, wt, norm_w.reshape(1, DIM_), bg.reshape(1, 2 * H_))

    ak = k.reshape(NB_, BLK_, KVH_, DH_).transpose(2, 0, 1, 3).reshape(
        KVH_ * NB_, HID_)
    av = v.reshape(NB_, BLK_, KVH_, DH_).transpose(2, 0, 1, 3).reshape(
        KVH_ * NB_, HID_)
    kp = k_pos.reshape(KVH_, HID_)
    vp = v_pos.reshape(KVH_, HID_)
    nt = HID_ // _HT
    ck_b, cv_b = pl.pallas_call(
        _comp_body,
        grid=(nt,),
        in_specs=[pl.BlockSpec((KVH_ * NB_, HID_), lambda t: (0, 0)),
                  pl.BlockSpec((KVH_ * NB_, HID_), lambda t: (0, 0)),
                  pl.BlockSpec((KVH_, HID_), lambda t: (0, 0)),
                  pl.BlockSpec((KVH_, HID_), lambda t: (0, 0)),
                  pl.BlockSpec((HID_, _HT), lambda t: (0, t)),
                  pl.BlockSpec((1, _HT), lambda t: (0, t)),
                  pl.BlockSpec((_HT, DH_), lambda t: (t, 0)),
                  pl.BlockSpec((HID_, _HT), lambda t: (0, t)),
                  pl.BlockSpec((1, _HT), lambda t: (0, t)),
                  pl.BlockSpec((_HT, DH_), lambda t: (t, 0)),
                  pl.BlockSpec((1, DH_), lambda t: (0, 0)),
                  pl.BlockSpec((1, DH_), lambda t: (0, 0))],
        out_specs=[pl.BlockSpec((KVH_ * NB_, DH_), lambda t: (0, 0)),
                   pl.BlockSpec((KVH_ * NB_, DH_), lambda t: (0, 0))],
        out_shape=[jax.ShapeDtypeStruct((KVH_ * NB_, DH_), jnp.float32),
                   jax.ShapeDtypeStruct((KVH_ * NB_, DH_), jnp.float32)],
    )(ak, av, kp, vp, Wk1.T, bk1.reshape(1, HID_), Wk2.T,
      Wv1.T, bv1.reshape(1, HID_), Wv2.T,
      bk2.reshape(1, DH_), bv2.reshape(1, DH_))

    ck = jnp.concatenate([mem_kv[0], ck_b.reshape(KVH_, NB_, DH_)], axis=1)
    cv = jnp.concatenate([mem_kv[1], cv_b.reshape(KVH_, NB_, DH_)], axis=1)
    kvp = jnp.concatenate([k.reshape(S_, KVH_, DH_).transpose(1, 0, 2),
                           v.reshape(S_, KVH_, DH_).transpose(1, 0, 2)],
                          axis=2)

    comp_n, fine_n = pl.pallas_call(
        _attn_body,
        grid=(KVH_,),
        in_specs=[
            pl.BlockSpec((S_, GQ_ * DH_), lambda h: (0, h)),
            pl.BlockSpec((1, S_, 2 * DH_), lambda h: (h, 0, 0)),
            pl.BlockSpec((1, NB_ + NMEM_, DH_), lambda h: (h, 0, 0)),
            pl.BlockSpec((1, NB_ + NMEM_, DH_), lambda h: (h, 0, 0)),
            pl.BlockSpec((S_, 4 * DH_), lambda h: (0, 0)),
            pl.BlockSpec((BLK_, BLK_), lambda h: (0, 0)),
        ],
        out_specs=[pl.BlockSpec((S_, GQ_ * DH_), lambda h: (0, h)),
                   pl.BlockSpec((S_, GQ_ * DH_), lambda h: (0, h))],
        out_shape=[jax.ShapeDtypeStruct((S_, H_ * DH_), jnp.float32),
                   jax.ShapeDtypeStruct((S_, H_ * DH_), jnp.float32)],
        scratch_shapes=[pltpu.VMEM((S_, _AUG), jnp.float32)],
    )(q, kvp, ck, cv, aux, tri)

    e0 = jnp.zeros((2 * H_, H_ * DH_), jnp.float32)
    hh = jnp.arange(H_ * DH_) // DH_
    e0 = e0.at[2 * hh, jnp.arange(H_ * DH_)].set(1.0)
    e1 = jnp.zeros((2 * H_, H_ * DH_), jnp.float32)
    e1 = e1.at[2 * hh + 1, jnp.arange(H_ * DH_)].set(1.0)

    out = pl.pallas_call(
        _proj_body,
        grid=(na,),
        in_specs=[row_spec(H_ * DH_), row_spec(H_ * DH_), row_spec(2 * H_),
                  pl.BlockSpec((2 * H_, H_ * DH_), lambda i: (0, 0)),
                  pl.BlockSpec((2 * H_, H_ * DH_), lambda i: (0, 0)),
                  pl.BlockSpec((H_ * DH_, DIM_), lambda i: (0, 0))],
        out_specs=row_spec(DIM_),
        out_shape=jax.ShapeDtypeStruct((S_, DIM_), jnp.float32),
    )(comp_n, fine_n, g, e0, e1, Wo.T)
    return out.reshape(1, S_, DIM_)


# speedup vs baseline: 3.2498x; 1.1173x over previous
import jax
import jax.numpy as jnp
import numpy as np
from jax.experimental import pallas as pl
from jax.experimental.pallas import tpu as pltpu

S_ = 2048
DIM_ = 1024
DH_ = 64
H_ = 16
KVH_ = 4
GQ_ = H_ // KVH_
BLK_ = 64
NB_ = S_ // BLK_
NSEL_ = 8
NMEM_ = 1
HID_ = BLK_ * DH_
SCALE_ = DH_ ** -0.5
MNEG_ = float(-jnp.finfo(jnp.float32).max)
EPS_ = float(jnp.finfo(jnp.float32).eps)

_RA = 256
_HT = 512
_GP = 256
_NG = S_ // _GP
_AUG = DH_ + NB_

_posn = np.arange(S_, dtype=np.float64)
_invn = 1.0 / (10000.0 ** (np.arange(0, DH_, 2, dtype=np.float64) / DH_))
_angn = _posn[:, None] * _invn[None, :]
_cosn = np.repeat(np.cos(_angn), 2, axis=-1).astype(np.float32)
_sinn = np.repeat(np.sin(_angn), 2, axis=-1).astype(np.float32)
_mtn = np.repeat(np.eye(NB_, dtype=np.float32), BLK_, axis=1).T
_AUXN = np.concatenate(
    [_cosn, _sinn, _mtn, np.zeros((S_, 2 * DH_ - NB_), np.float32)], axis=1)
_tin = np.arange(BLK_)
_TRIN = np.where(_tin[None, :] <= _tin[:, None], 0.0, -1e30).astype(np.float32)
_hhn = np.arange(H_ * DH_) // DH_
_E0N = np.zeros((2 * H_, H_ * DH_), np.float32)
_E0N[2 * _hhn, np.arange(H_ * DH_)] = 1.0
_E1N = np.zeros((2 * H_, H_ * DH_), np.float32)
_E1N[2 * _hhn + 1, np.arange(H_ * DH_)] = 1.0


def _rot(t, cos, sin):
    a = jnp.concatenate([t[:, -1:], t[:, :-1]], axis=1)
    b = jnp.concatenate([t[:, 1:], t[:, :1]], axis=1)
    lane = jax.lax.broadcasted_iota(jnp.int32, t.shape, 1)
    rh = jnp.where(lane % 2 == 0, -b, a)
    return t * cos + rh * sin


def _qkv_body(x_ref, wt_ref, nw_ref, bg_ref, q_ref, k_ref, v_ref, g_ref):
    x = x_ref[...]
    ms = jnp.mean(x * x, axis=1, keepdims=True)
    x = x * jax.lax.rsqrt(ms + EPS_) * nw_ref[...]
    y = jnp.dot(x, wt_ref[...], preferred_element_type=jnp.float32)
    q_ref[...] = y[:, :H_ * DH_]
    for h in range(KVH_):
        k_ref[h, :, :] = y[:, (H_ + h) * DH_:(H_ + h + 1) * DH_]
        v_ref[h, :, :] = y[:, (H_ + KVH_ + h) * DH_:(H_ + KVH_ + h + 1) * DH_]
    g_ref[...] = jax.nn.sigmoid(y[:, (H_ + 2 * KVH_) * DH_:] + bg_ref[...])


def _comp_body(ak_ref, av_ref, kp_ref, vp_ref, w1k_ref, b1k_ref, w2k_ref,
               w1v_ref, b1v_ref, w2v_ref, bk2_ref, bv2_ref, ck_ref, cv_ref):
    t = pl.program_id(0)

    @pl.when(t == 0)
    def _():
        ck_ref[...] = jnp.broadcast_to(bk2_ref[...], ck_ref.shape)
        cv_ref[...] = jnp.broadcast_to(bv2_ref[...], cv_ref.shape)

    ak = (ak_ref[...].reshape(KVH_, NB_, HID_)
          + kp_ref[...][:, None, :]).reshape(KVH_ * NB_, HID_)
    av = (av_ref[...].reshape(KVH_, NB_, HID_)
          + vp_ref[...][:, None, :]).reshape(KVH_ * NB_, HID_)
    hk = jnp.maximum(
        jnp.dot(ak, w1k_ref[...], preferred_element_type=jnp.float32)
        + b1k_ref[...], 0.0)
    hv = jnp.maximum(
        jnp.dot(av, w1v_ref[...], preferred_element_type=jnp.float32)
        + b1v_ref[...], 0.0)
    ck_ref[...] += jnp.dot(hk, w2k_ref[...], preferred_element_type=jnp.float32)
    cv_ref[...] += jnp.dot(hv, w2v_ref[...], preferred_element_type=jnp.float32)


def _attn_body(q_ref, k_ref, v_ref, ck_ref, cv_ref, aux_ref, tri_ref,
               comp_ref, fine_ref, kr_ref):
    kr_ref[...] = jnp.concatenate(
        [_rot(k_ref[0], aux_ref[:, :DH_], aux_ref[:, DH_:2 * DH_]),
         aux_ref[:, 2 * DH_:2 * DH_ + NB_]], axis=1)
    ckt = ck_ref[0]
    cvt = cv_ref[0]
    for qg in range(_NG):
        lo = qg * _GP
        span = lo + _GP
        cosg = aux_ref[lo:span, :DH_]
        sing = aux_ref[lo:span, DH_:2 * DH_]
        csims = []
        for g in range(GQ_):
            qpg = q_ref[lo:span, g * DH_:(g + 1) * DH_]
            csims.append(jax.lax.dot_general(
                qpg, ckt, (((1,), (1,)), ((), ())),
                preferred_element_type=jnp.float32) * SCALE_)
        ri = jax.lax.broadcasted_iota(jnp.int32, csims[0].shape, 0)
        ci = jax.lax.broadcasted_iota(jnp.int32, csims[0].shape, 1)
        cvis = ci < (lo + ri) // BLK_ + 1
        imp = jnp.zeros(csims[0].shape, jnp.float32)
        for g in range(GQ_):
            cs = jnp.where(cvis, csims[g], MNEG_)
            imp = imp + cs
            cmax = jnp.max(cs, axis=1, keepdims=True)
            cp = jnp.exp(cs - cmax)
            cattn = cp / jnp.sum(cp, axis=1, keepdims=True)
            comp_ref[lo:span, g * DH_:(g + 1) * DH_] = jnp.dot(
                cattn, cvt, preferred_element_type=jnp.float32)

        imp = imp[:, NMEM_:] * (1.0 / GQ_)
        m2 = jnp.maximum(jnp.max(imp, axis=1, keepdims=True), -1e3)
        e = jnp.exp(imp - m2)
        probs = e / (jnp.sum(e, axis=1, keepdims=True) + jnp.exp(-1e3 - m2))

        work = probs.T
        rowi = jax.lax.broadcasted_iota(jnp.int32, work.shape, 0)
        sel_t = jnp.full(work.shape, -1e30, jnp.float32)
        for _ in range(NSEL_):
            m = jnp.max(work, axis=0, keepdims=True)
            cand = work == m
            idxv = jnp.where(cand, rowi, NB_ * 2)
            pick = rowi == jnp.min(idxv, axis=0, keepdims=True)
            sel_t = jnp.where(pick & (m > 1e-10), 0.0, sel_t)
            work = jnp.where(pick, -1.0, work)
        sel_pos = sel_t.T

        for g in range(GQ_):
            qrg = _rot(q_ref[lo:span, g * DH_:(g + 1) * DH_],
                       cosg, sing) * SCALE_
            aq = jnp.concatenate([qrg, sel_pos], axis=1)
            s_all = jax.lax.dot_general(
                aq, kr_ref[0:span, :], (((1,), (1,)), ((), ())),
                preferred_element_type=jnp.float32)
            ods = []
            for b in range(GQ_):
                qb_ = qrg[b * BLK_:(b + 1) * BLK_, :]
                kb_ = kr_ref[lo + b * BLK_:lo + (b + 1) * BLK_, :DH_]
                ods.append(jax.lax.dot_general(
                    qb_, kb_, (((1,), (1,)), ((), ())),
                    preferred_element_type=jnp.float32) + tri_ref[...])
            s_od = jnp.concatenate(ods, axis=0)
            mm = jnp.maximum(jnp.max(s_all, axis=1, keepdims=True),
                             jnp.max(s_od, axis=1, keepdims=True))
            p_all = jnp.exp(s_all - mm)
            p_od = jnp.exp(s_od - mm)
            l_f = (jnp.sum(p_all, axis=1, keepdims=True)
                   + jnp.sum(p_od, axis=1, keepdims=True))
            acc = jnp.dot(p_all, v_ref[0, 0:span, :],
                          preferred_element_type=jnp.float32)
            oacc = []
            for b in range(GQ_):
                oacc.append(jnp.dot(
                    p_od[b * BLK_:(b + 1) * BLK_, :],
                    v_ref[0, lo + b * BLK_:lo + (b + 1) * BLK_, :],
                    preferred_element_type=jnp.float32))
            acc = acc + jnp.concatenate(oacc, axis=0)
            fine_ref[lo:span, g * DH_:(g + 1) * DH_] = acc / l_f


def _proj_body(comp_ref, fine_ref, g_ref, e0_ref, e1_ref, w_ref, out_ref):
    g = g_ref[...]
    g0 = jnp.dot(g, e0_ref[...], preferred_element_type=jnp.float32)
    g1 = jnp.dot(g, e1_ref[...], preferred_element_type=jnp.float32)
    oh = g0 * comp_ref[...] + g1 * fine_ref[...]
    out_ref[...] = jnp.dot(oh, w_ref[...], preferred_element_type=jnp.float32)


def kernel(inp, norm_w, W_qkv, k_pos, v_pos, mem_kv, Wk1, bk1, Wk2, bk2,
           Wv1, bv1, Wv2, bv2, Wg, bg, Wo):
    x2 = inp.reshape(S_, DIM_)
    wt = jnp.concatenate([W_qkv, Wg], axis=0).T
    aux = jnp.asarray(_AUXN)
    tri = jnp.asarray(_TRIN)

    na = S_ // _RA
    row_spec = lambda w: pl.BlockSpec((_RA, w), lambda i: (i, 0))
    hs_spec = pl.BlockSpec((KVH_, _RA, DH_), lambda i: (0, i, 0))
    q, k, v, g = pl.pallas_call(
        _qkv_body,
        grid=(na,),
        in_specs=[row_spec(DIM_),
                  pl.BlockSpec((DIM_, (H_ + 2 * KVH_) * DH_ + 2 * H_),
                               lambda i: (0, 0)),
                  pl.BlockSpec((1, DIM_), lambda i: (0, 0)),
                  pl.BlockSpec((1, 2 * H_), lambda i: (0, 0))],
        out_specs=[row_spec(H_ * DH_), hs_spec, hs_spec, row_spec(2 * H_)],
        out_shape=[jax.ShapeDtypeStruct((S_, H_ * DH_), jnp.float32),
                   jax.ShapeDtypeStruct((KVH_, S_, DH_), jnp.float32),
                   jax.ShapeDtypeStruct((KVH_, S_, DH_), jnp.float32),
                   jax.ShapeDtypeStruct((S_, 2 * H_), jnp.float32)],
    )(x2, wt, norm_w.reshape(1, DIM_), bg.reshape(1, 2 * H_))

    ak = k.reshape(KVH_ * NB_, HID_)
    av = v.reshape(KVH_ * NB_, HID_)
    kp = k_pos.reshape(KVH_, HID_)
    vp = v_pos.reshape(KVH_, HID_)
    nt = HID_ // _HT
    ck_b, cv_b = pl.pallas_call(
        _comp_body,
        grid=(nt,),
        in_specs=[pl.BlockSpec((KVH_ * NB_, HID_), lambda t: (0, 0)),
                  pl.BlockSpec((KVH_ * NB_, HID_), lambda t: (0, 0)),
                  pl.BlockSpec((KVH_, HID_), lambda t: (0, 0)),
                  pl.BlockSpec((KVH_, HID_), lambda t: (0, 0)),
                  pl.BlockSpec((HID_, _HT), lambda t: (0, t)),
                  pl.BlockSpec((1, _HT), lambda t: (0, t)),
                  pl.BlockSpec((_HT, DH_), lambda t: (t, 0)),
                  pl.BlockSpec((HID_, _HT), lambda t: (0, t)),
                  pl.BlockSpec((1, _HT), lambda t: (0, t)),
                  pl.BlockSpec((_HT, DH_), lambda t: (t, 0)),
                  pl.BlockSpec((1, DH_), lambda t: (0, 0)),
                  pl.BlockSpec((1, DH_), lambda t: (0, 0))],
        out_specs=[pl.BlockSpec((KVH_ * NB_, DH_), lambda t: (0, 0)),
                   pl.BlockSpec((KVH_ * NB_, DH_), lambda t: (0, 0))],
        out_shape=[jax.ShapeDtypeStruct((KVH_ * NB_, DH_), jnp.float32),
                   jax.ShapeDtypeStruct((KVH_ * NB_, DH_), jnp.float32)],
    )(ak, av, kp, vp, Wk1.T, bk1.reshape(1, HID_), Wk2.T,
      Wv1.T, bv1.reshape(1, HID_), Wv2.T,
      bk2.reshape(1, DH_), bv2.reshape(1, DH_))

    ck = jnp.concatenate([mem_kv[0], ck_b.reshape(KVH_, NB_, DH_)], axis=1)
    cv = jnp.concatenate([mem_kv[1], cv_b.reshape(KVH_, NB_, DH_)], axis=1)

    comp_n, fine_n = pl.pallas_call(
        _attn_body,
        grid=(KVH_,),
        in_specs=[
            pl.BlockSpec((S_, GQ_ * DH_), lambda h: (0, h)),
            pl.BlockSpec((1, S_, DH_), lambda h: (h, 0, 0)),
            pl.BlockSpec((1, S_, DH_), lambda h: (h, 0, 0)),
            pl.BlockSpec((1, NB_ + NMEM_, DH_), lambda h: (h, 0, 0)),
            pl.BlockSpec((1, NB_ + NMEM_, DH_), lambda h: (h, 0, 0)),
            pl.BlockSpec((S_, 4 * DH_), lambda h: (0, 0)),
            pl.BlockSpec((BLK_, BLK_), lambda h: (0, 0)),
        ],
        out_specs=[pl.BlockSpec((S_, GQ_ * DH_), lambda h: (0, h)),
                   pl.BlockSpec((S_, GQ_ * DH_), lambda h: (0, h))],
        out_shape=[jax.ShapeDtypeStruct((S_, H_ * DH_), jnp.float32),
                   jax.ShapeDtypeStruct((S_, H_ * DH_), jnp.float32)],
        scratch_shapes=[pltpu.VMEM((S_, _AUG), jnp.float32)],
    )(q, k, v, ck, cv, aux, tri)

    e0 = jnp.asarray(_E0N)
    e1 = jnp.asarray(_E1N)

    out = pl.pallas_call(
        _proj_body,
        grid=(na,),
        in_specs=[row_spec(H_ * DH_), row_spec(H_ * DH_), row_spec(2 * H_),
                  pl.BlockSpec((2 * H_, H_ * DH_), lambda i: (0, 0)),
                  pl.BlockSpec((2 * H_, H_ * DH_), lambda i: (0, 0)),
                  pl.BlockSpec((H_ * DH_, DIM_), lambda i: (0, 0))],
        out_specs=row_spec(DIM_),
        out_shape=jax.ShapeDtypeStruct((S_, DIM_), jnp.float32),
    )(comp_n, fine_n, g, e0, e1, Wo.T)
    return out.reshape(1, S_, DIM_)


# speedup vs baseline: 4.3687x; 1.3443x over previous
import jax
import jax.numpy as jnp
import numpy as np
from jax.experimental import pallas as pl
from jax.experimental.pallas import tpu as pltpu

S_ = 2048
DIM_ = 1024
DH_ = 64
H_ = 16
KVH_ = 4
GQ_ = H_ // KVH_
BLK_ = 64
NB_ = S_ // BLK_
NSEL_ = 8
NMEM_ = 1
HID_ = BLK_ * DH_
SCALE_ = DH_ ** -0.5
MNEG_ = float(-jnp.finfo(jnp.float32).max)
EPS_ = float(jnp.finfo(jnp.float32).eps)

_RA = 256
_HT = 512
_GP = 256
_NG = S_ // _GP
_AUG = DH_ + NB_

_posn = np.arange(S_, dtype=np.float64)
_invn = 1.0 / (10000.0 ** (np.arange(0, DH_, 2, dtype=np.float64) / DH_))
_angn = _posn[:, None] * _invn[None, :]
_cosn = np.repeat(np.cos(_angn), 2, axis=-1).astype(np.float32)
_sinn = np.repeat(np.sin(_angn), 2, axis=-1).astype(np.float32)
_mtn = np.repeat(np.eye(NB_, dtype=np.float32), BLK_, axis=1).T
_AUXN = np.concatenate(
    [_cosn, _sinn, _mtn, np.zeros((S_, 2 * DH_ - NB_), np.float32)], axis=1)
_tin = np.arange(BLK_)
_TRIN = np.where(_tin[None, :] <= _tin[:, None], 0.0, -1e30).astype(np.float32)
_hhn = np.arange(H_ * DH_) // DH_
_E0N = np.zeros((2 * H_, H_ * DH_), np.float32)
_E0N[2 * _hhn, np.arange(H_ * DH_)] = 1.0
_E1N = np.zeros((2 * H_, H_ * DH_), np.float32)
_E1N[2 * _hhn + 1, np.arange(H_ * DH_)] = 1.0


def _rot(t, cos, sin):
    a = jnp.concatenate([t[:, -1:], t[:, :-1]], axis=1)
    b = jnp.concatenate([t[:, 1:], t[:, :1]], axis=1)
    lane = jax.lax.broadcasted_iota(jnp.int32, t.shape, 1)
    rh = jnp.where(lane % 2 == 0, -b, a)
    return t * cos + rh * sin


def _qkv_body(x_ref, wt_ref, wg_ref, nw_ref, bg_ref, q_ref, k_ref, v_ref,
              g_ref):
    x = x_ref[...]
    ms = jnp.mean(x * x, axis=1, keepdims=True)
    x = x * jax.lax.rsqrt(ms + EPS_) * nw_ref[...]
    y = jax.lax.dot_general(x, wt_ref[...], (((1,), (1,)), ((), ())),
                            preferred_element_type=jnp.float32)
    gr = jax.lax.dot_general(x, wg_ref[...], (((1,), (1,)), ((), ())),
                             preferred_element_type=jnp.float32)
    q_ref[...] = y[:, :H_ * DH_]
    for h in range(KVH_):
        k_ref[h, :, :] = y[:, (H_ + h) * DH_:(H_ + h + 1) * DH_]
        v_ref[h, :, :] = y[:, (H_ + KVH_ + h) * DH_:(H_ + KVH_ + h + 1) * DH_]
    g_ref[...] = jax.nn.sigmoid(gr + bg_ref[...])


def _comp_body(ak_ref, av_ref, kp_ref, vp_ref, w1k_ref, b1k_ref, w2k_ref,
               w1v_ref, b1v_ref, w2v_ref, bk2_ref, bv2_ref, ck_ref, cv_ref):
    t = pl.program_id(0)

    @pl.when(t == 0)
    def _():
        ck_ref[...] = jnp.broadcast_to(bk2_ref[...], ck_ref.shape)
        cv_ref[...] = jnp.broadcast_to(bv2_ref[...], cv_ref.shape)

    ak = (ak_ref[...].reshape(KVH_, NB_, HID_)
          + kp_ref[...][:, None, :]).reshape(KVH_ * NB_, HID_)
    av = (av_ref[...].reshape(KVH_, NB_, HID_)
          + vp_ref[...][:, None, :]).reshape(KVH_ * NB_, HID_)
    hk = jnp.maximum(
        jax.lax.dot_general(ak, w1k_ref[...], (((1,), (1,)), ((), ())),
                            preferred_element_type=jnp.float32)
        + b1k_ref[...], 0.0)
    hv = jnp.maximum(
        jax.lax.dot_general(av, w1v_ref[...], (((1,), (1,)), ((), ())),
                            preferred_element_type=jnp.float32)
        + b1v_ref[...], 0.0)
    ck_ref[...] += jax.lax.dot_general(
        hk, w2k_ref[...], (((1,), (1,)), ((), ())),
        preferred_element_type=jnp.float32)
    cv_ref[...] += jax.lax.dot_general(
        hv, w2v_ref[...], (((1,), (1,)), ((), ())),
        preferred_element_type=jnp.float32)


def _attn_body(q_ref, k_ref, v_ref, ck_ref, cv_ref, aux_ref, tri_ref,
               comp_ref, fine_ref, kr_ref):
    kr_ref[...] = jnp.concatenate(
        [_rot(k_ref[0], aux_ref[:, :DH_], aux_ref[:, DH_:2 * DH_]),
         aux_ref[:, 2 * DH_:2 * DH_ + NB_]], axis=1)
    ckt = ck_ref[0]
    cvt = cv_ref[0]
    for qg in range(_NG):
        lo = qg * _GP
        span = lo + _GP
        cosg = aux_ref[lo:span, :DH_]
        sing = aux_ref[lo:span, DH_:2 * DH_]
        csims = []
        for g in range(GQ_):
            qpg = q_ref[lo:span, g * DH_:(g + 1) * DH_]
            csims.append(jax.lax.dot_general(
                qpg, ckt, (((1,), (1,)), ((), ())),
                preferred_element_type=jnp.float32) * SCALE_)
        ri = jax.lax.broadcasted_iota(jnp.int32, csims[0].shape, 0)
        ci = jax.lax.broadcasted_iota(jnp.int32, csims[0].shape, 1)
        cvis = ci < (lo + ri) // BLK_ + 1
        imp = jnp.zeros(csims[0].shape, jnp.float32)
        for g in range(GQ_):
            cs = jnp.where(cvis, csims[g], MNEG_)
            imp = imp + cs
            cmax = jnp.max(cs, axis=1, keepdims=True)
            cp = jnp.exp(cs - cmax)
            cattn = cp / jnp.sum(cp, axis=1, keepdims=True)
            comp_ref[lo:span, g * DH_:(g + 1) * DH_] = jnp.dot(
                cattn, cvt, preferred_element_type=jnp.float32)

        imp = imp[:, NMEM_:] * (1.0 / GQ_)
        m2 = jnp.maximum(jnp.max(imp, axis=1, keepdims=True), -1e3)
        e = jnp.exp(imp - m2)
        probs = e / (jnp.sum(e, axis=1, keepdims=True) + jnp.exp(-1e3 - m2))

        work = probs.T
        rowi = jax.lax.broadcasted_iota(jnp.int32, work.shape, 0)
        sel_t = jnp.full(work.shape, -1e30, jnp.float32)
        for _ in range(NSEL_):
            m = jnp.max(work, axis=0, keepdims=True)
            cand = work == m
            idxv = jnp.where(cand, rowi, NB_ * 2)
            pick = rowi == jnp.min(idxv, axis=0, keepdims=True)
            sel_t = jnp.where(pick & (m > 1e-10), 0.0, sel_t)
            work = jnp.where(pick, -1.0, work)
        sel_pos = sel_t.T

        for g in range(GQ_):
            qrg = _rot(q_ref[lo:span, g * DH_:(g + 1) * DH_],
                       cosg, sing) * SCALE_
            aq = jnp.concatenate([qrg, sel_pos], axis=1)
            s_all = jax.lax.dot_general(
                aq, kr_ref[0:span, :], (((1,), (1,)), ((), ())),
                preferred_element_type=jnp.float32)
            ods = []
            for b in range(GQ_):
                qb_ = qrg[b * BLK_:(b + 1) * BLK_, :]
                kb_ = kr_ref[lo + b * BLK_:lo + (b + 1) * BLK_, :DH_]
                ods.append(jax.lax.dot_general(
                    qb_, kb_, (((1,), (1,)), ((), ())),
                    preferred_element_type=jnp.float32) + tri_ref[...])
            s_od = jnp.concatenate(ods, axis=0)
            mm = jnp.maximum(jnp.max(s_all, axis=1, keepdims=True),
                             jnp.max(s_od, axis=1, keepdims=True))
            p_all = jnp.exp(s_all - mm)
            p_od = jnp.exp(s_od - mm)
            l_f = (jnp.sum(p_all, axis=1, keepdims=True)
                   + jnp.sum(p_od, axis=1, keepdims=True))
            acc = jnp.dot(p_all, v_ref[0, 0:span, :],
                          preferred_element_type=jnp.float32)
            oacc = []
            for b in range(GQ_):
                oacc.append(jnp.dot(
                    p_od[b * BLK_:(b + 1) * BLK_, :],
                    v_ref[0, lo + b * BLK_:lo + (b + 1) * BLK_, :],
                    preferred_element_type=jnp.float32))
            acc = acc + jnp.concatenate(oacc, axis=0)
            fine_ref[lo:span, g * DH_:(g + 1) * DH_] = acc / l_f


def _proj_body(comp_ref, fine_ref, g_ref, e0_ref, e1_ref, w_ref, out_ref):
    g = g_ref[...]
    g0 = jnp.dot(g, e0_ref[...], preferred_element_type=jnp.float32)
    g1 = jnp.dot(g, e1_ref[...], preferred_element_type=jnp.float32)
    oh = g0 * comp_ref[...] + g1 * fine_ref[...]
    out_ref[...] = jax.lax.dot_general(
        oh, w_ref[...], (((1,), (1,)), ((), ())),
        preferred_element_type=jnp.float32)


def kernel(inp, norm_w, W_qkv, k_pos, v_pos, mem_kv, Wk1, bk1, Wk2, bk2,
           Wv1, bv1, Wv2, bv2, Wg, bg, Wo):
    x2 = inp.reshape(S_, DIM_)
    aux = jnp.asarray(_AUXN)
    tri = jnp.asarray(_TRIN)

    na = S_ // _RA
    row_spec = lambda w: pl.BlockSpec((_RA, w), lambda i: (i, 0))
    hs_spec = pl.BlockSpec((KVH_, _RA, DH_), lambda i: (0, i, 0))
    q, k, v, g = pl.pallas_call(
        _qkv_body,
        grid=(na,),
        in_specs=[row_spec(DIM_),
                  pl.BlockSpec(((H_ + 2 * KVH_) * DH_, DIM_),
                               lambda i: (0, 0)),
                  pl.BlockSpec((2 * H_, DIM_), lambda i: (0, 0)),
                  pl.BlockSpec((1, DIM_), lambda i: (0, 0)),
                  pl.BlockSpec((1, 2 * H_), lambda i: (0, 0))],
        out_specs=[row_spec(H_ * DH_), hs_spec, hs_spec, row_spec(2 * H_)],
        out_shape=[jax.ShapeDtypeStruct((S_, H_ * DH_), jnp.float32),
                   jax.ShapeDtypeStruct((KVH_, S_, DH_), jnp.float32),
                   jax.ShapeDtypeStruct((KVH_, S_, DH_), jnp.float32),
                   jax.ShapeDtypeStruct((S_, 2 * H_), jnp.float32)],
    )(x2, W_qkv, Wg, norm_w.reshape(1, DIM_), bg.reshape(1, 2 * H_))

    ak = k.reshape(KVH_ * NB_, HID_)
    av = v.reshape(KVH_ * NB_, HID_)
    kp = k_pos.reshape(KVH_, HID_)
    vp = v_pos.reshape(KVH_, HID_)
    nt = HID_ // _HT
    ck_b, cv_b = pl.pallas_call(
        _comp_body,
        grid=(nt,),
        in_specs=[pl.BlockSpec((KVH_ * NB_, HID_), lambda t: (0, 0)),
                  pl.BlockSpec((KVH_ * NB_, HID_), lambda t: (0, 0)),
                  pl.BlockSpec((KVH_, HID_), lambda t: (0, 0)),
                  pl.BlockSpec((KVH_, HID_), lambda t: (0, 0)),
                  pl.BlockSpec((_HT, HID_), lambda t: (t, 0)),
                  pl.BlockSpec((1, _HT), lambda t: (0, t)),
                  pl.BlockSpec((DH_, _HT), lambda t: (0, t)),
                  pl.BlockSpec((_HT, HID_), lambda t: (t, 0)),
                  pl.BlockSpec((1, _HT), lambda t: (0, t)),
                  pl.BlockSpec((DH_, _HT), lambda t: (0, t)),
                  pl.BlockSpec((1, DH_), lambda t: (0, 0)),
                  pl.BlockSpec((1, DH_), lambda t: (0, 0))],
        out_specs=[pl.BlockSpec((KVH_ * NB_, DH_), lambda t: (0, 0)),
                   pl.BlockSpec((KVH_ * NB_, DH_), lambda t: (0, 0))],
        out_shape=[jax.ShapeDtypeStruct((KVH_ * NB_, DH_), jnp.float32),
                   jax.ShapeDtypeStruct((KVH_ * NB_, DH_), jnp.float32)],
    )(ak, av, kp, vp, Wk1, bk1.reshape(1, HID_), Wk2,
      Wv1, bv1.reshape(1, HID_), Wv2,
      bk2.reshape(1, DH_), bv2.reshape(1, DH_))

    ck = jnp.concatenate([mem_kv[0], ck_b.reshape(KVH_, NB_, DH_)], axis=1)
    cv = jnp.concatenate([mem_kv[1], cv_b.reshape(KVH_, NB_, DH_)], axis=1)

    comp_n, fine_n = pl.pallas_call(
        _attn_body,
        grid=(KVH_,),
        in_specs=[
            pl.BlockSpec((S_, GQ_ * DH_), lambda h: (0, h)),
            pl.BlockSpec((1, S_, DH_), lambda h: (h, 0, 0)),
            pl.BlockSpec((1, S_, DH_), lambda h: (h, 0, 0)),
            pl.BlockSpec((1, NB_ + NMEM_, DH_), lambda h: (h, 0, 0)),
            pl.BlockSpec((1, NB_ + NMEM_, DH_), lambda h: (h, 0, 0)),
            pl.BlockSpec((S_, 4 * DH_), lambda h: (0, 0)),
            pl.BlockSpec((BLK_, BLK_), lambda h: (0, 0)),
        ],
        out_specs=[pl.BlockSpec((S_, GQ_ * DH_), lambda h: (0, h)),
                   pl.BlockSpec((S_, GQ_ * DH_), lambda h: (0, h))],
        out_shape=[jax.ShapeDtypeStruct((S_, H_ * DH_), jnp.float32),
                   jax.ShapeDtypeStruct((S_, H_ * DH_), jnp.float32)],
        scratch_shapes=[pltpu.VMEM((S_, _AUG), jnp.float32)],
    )(q, k, v, ck, cv, aux, tri)

    e0 = jnp.asarray(_E0N)
    e1 = jnp.asarray(_E1N)

    out = pl.pallas_call(
        _proj_body,
        grid=(na,),
        in_specs=[row_spec(H_ * DH_), row_spec(H_ * DH_), row_spec(2 * H_),
                  pl.BlockSpec((2 * H_, H_ * DH_), lambda i: (0, 0)),
                  pl.BlockSpec((2 * H_, H_ * DH_), lambda i: (0, 0)),
                  pl.BlockSpec((H_ * DH_, DIM_), lambda i: (0, 0))],
        out_specs=row_spec(DIM_),
        out_shape=jax.ShapeDtypeStruct((S_, DIM_), jnp.float32),
    )(comp_n, fine_n, g, e0, e1, Wo)
    return out.reshape(1, S_, DIM_)


# speedup vs baseline: 6.2647x; 1.4340x over previous
import jax
import jax.numpy as jnp
import numpy as np
from jax.experimental import pallas as pl
from jax.experimental.pallas import tpu as pltpu

S_ = 2048
DIM_ = 1024
DH_ = 64
H_ = 16
KVH_ = 4
GQ_ = H_ // KVH_
BLK_ = 64
NB_ = S_ // BLK_
NSEL_ = 8
NMEM_ = 1
HID_ = BLK_ * DH_
SCALE_ = DH_ ** -0.5
MNEG_ = float(-jnp.finfo(jnp.float32).max)
EPS_ = float(jnp.finfo(jnp.float32).eps)

_RA = 256
_HT = 512
_GP = 256
_NG = S_ // _GP
_AUG = DH_ + NB_

_posn = np.arange(S_, dtype=np.float64)
_invn = 1.0 / (10000.0 ** (np.arange(0, DH_, 2, dtype=np.float64) / DH_))
_angn = _posn[:, None] * _invn[None, :]
_cosn = np.repeat(np.cos(_angn), 2, axis=-1).astype(np.float32)
_sinn = np.repeat(np.sin(_angn), 2, axis=-1).astype(np.float32)
_mtn = np.repeat(np.eye(NB_, dtype=np.float32), BLK_, axis=1).T
_AUXN = np.concatenate(
    [_cosn, _sinn, _mtn, np.zeros((S_, 2 * DH_ - NB_), np.float32)], axis=1)
_tin = np.arange(BLK_)
_TRIN = np.where(_tin[None, :] <= _tin[:, None], 0.0, -1e30).astype(np.float32)
_hhn = np.arange(H_ * DH_) // DH_
_E0N = np.zeros((2 * H_, H_ * DH_), np.float32)
_E0N[2 * _hhn, np.arange(H_ * DH_)] = 1.0
_E1N = np.zeros((2 * H_, H_ * DH_), np.float32)
_E1N[2 * _hhn + 1, np.arange(H_ * DH_)] = 1.0


def _rot(t, cos, sin):
    a = jnp.concatenate([t[:, -1:], t[:, :-1]], axis=1)
    b = jnp.concatenate([t[:, 1:], t[:, :1]], axis=1)
    lane = jax.lax.broadcasted_iota(jnp.int32, t.shape, 1)
    rh = jnp.where(lane % 2 == 0, -b, a)
    return t * cos + rh * sin


def _qkv_body(x_ref, wt_ref, wg_ref, nw_ref, bg_ref, q_ref, k_ref, v_ref,
              g_ref):
    x = x_ref[...]
    ms = jnp.mean(x * x, axis=1, keepdims=True)
    x = x * jax.lax.rsqrt(ms + EPS_) * nw_ref[...]
    y = jax.lax.dot_general(x, wt_ref[...], (((1,), (1,)), ((), ())),
                            preferred_element_type=jnp.float32)
    gr = jax.lax.dot_general(x, wg_ref[...], (((1,), (1,)), ((), ())),
                             preferred_element_type=jnp.float32)
    q_ref[...] = y[:, :H_ * DH_]
    for h in range(KVH_):
        k_ref[h, :, :] = y[:, (H_ + h) * DH_:(H_ + h + 1) * DH_]
        v_ref[h, :, :] = y[:, (H_ + KVH_ + h) * DH_:(H_ + KVH_ + h + 1) * DH_]
    g_ref[...] = jax.nn.sigmoid(gr + bg_ref[...])


def _comp_body(ak_ref, av_ref, kp_ref, vp_ref, w1k_ref, b1k_ref, w2k_ref,
               w1v_ref, b1v_ref, w2v_ref, bk2_ref, bv2_ref, ck_ref, cv_ref):
    t = pl.program_id(0)

    @pl.when(t == 0)
    def _():
        ck_ref[...] = jnp.broadcast_to(bk2_ref[...], ck_ref.shape)
        cv_ref[...] = jnp.broadcast_to(bv2_ref[...], cv_ref.shape)

    ak = (ak_ref[...].reshape(KVH_, NB_, HID_)
          + kp_ref[...][:, None, :]).reshape(KVH_ * NB_, HID_)
    av = (av_ref[...].reshape(KVH_, NB_, HID_)
          + vp_ref[...][:, None, :]).reshape(KVH_ * NB_, HID_)
    hk = jnp.maximum(
        jax.lax.dot_general(ak, w1k_ref[...], (((1,), (1,)), ((), ())),
                            preferred_element_type=jnp.float32)
        + b1k_ref[...], 0.0)
    hv = jnp.maximum(
        jax.lax.dot_general(av, w1v_ref[...], (((1,), (1,)), ((), ())),
                            preferred_element_type=jnp.float32)
        + b1v_ref[...], 0.0)
    ck_ref[...] += jax.lax.dot_general(
        hk, w2k_ref[...], (((1,), (1,)), ((), ())),
        preferred_element_type=jnp.float32)
    cv_ref[...] += jax.lax.dot_general(
        hv, w2v_ref[...], (((1,), (1,)), ((), ())),
        preferred_element_type=jnp.float32)


def _attn_body(q_ref, k_ref, v_ref, ck_ref, cv_ref, aux_ref, tri_ref,
               comp_ref, fine_ref, kr_ref, va_ref):
    kr_ref[...] = jnp.concatenate(
        [_rot(k_ref[0], aux_ref[:, :DH_], aux_ref[:, DH_:2 * DH_]),
         aux_ref[:, 2 * DH_:2 * DH_ + NB_]], axis=1).astype(jnp.bfloat16)
    va_ref[...] = jnp.concatenate(
        [v_ref[0], jnp.ones((S_, 1), jnp.float32)], axis=1).astype(jnp.bfloat16)
    ckt = ck_ref[0]
    cvt = cv_ref[0]
    for qg in range(_NG):
        lo = qg * _GP
        span = lo + _GP
        cosg = aux_ref[lo:span, :DH_]
        sing = aux_ref[lo:span, DH_:2 * DH_]
        csims = []
        for g in range(GQ_):
            qpg = q_ref[lo:span, g * DH_:(g + 1) * DH_]
            csims.append(jax.lax.dot_general(
                qpg, ckt, (((1,), (1,)), ((), ())),
                preferred_element_type=jnp.float32) * SCALE_)
        ri = jax.lax.broadcasted_iota(jnp.int32, csims[0].shape, 0)
        ci = jax.lax.broadcasted_iota(jnp.int32, csims[0].shape, 1)
        cvis = ci < (lo + ri) // BLK_ + 1
        imp = jnp.zeros(csims[0].shape, jnp.float32)
        for g in range(GQ_):
            cs = jnp.where(cvis, csims[g], MNEG_)
            imp = imp + cs
            cmax = jnp.max(cs, axis=1, keepdims=True)
            cp = jnp.exp(cs - cmax)
            cattn = cp / jnp.sum(cp, axis=1, keepdims=True)
            comp_ref[lo:span, g * DH_:(g + 1) * DH_] = jnp.dot(
                cattn, cvt, preferred_element_type=jnp.float32)

        imp = imp[:, NMEM_:] * (1.0 / GQ_)
        m2 = jnp.maximum(jnp.max(imp, axis=1, keepdims=True), -1e3)
        e = jnp.exp(imp - m2)
        probs = e / (jnp.sum(e, axis=1, keepdims=True) + jnp.exp(-1e3 - m2))

        work = probs.T
        rowi = jax.lax.broadcasted_iota(jnp.int32, work.shape, 0)
        sel_t = jnp.full(work.shape, -1e30, jnp.float32)
        for _ in range(NSEL_):
            m = jnp.max(work, axis=0, keepdims=True)
            cand = work == m
            idxv = jnp.where(cand, rowi, NB_ * 2)
            pick = rowi == jnp.min(idxv, axis=0, keepdims=True)
            sel_t = jnp.where(pick & (m > 1e-10), 0.0, sel_t)
            work = jnp.where(pick, -1.0, work)
        sel_pos = sel_t.T

        for g in range(GQ_):
            qrg = (_rot(q_ref[lo:span, g * DH_:(g + 1) * DH_],
                        cosg, sing) * SCALE_)
            aq = jnp.concatenate(
                [qrg, sel_pos], axis=1).astype(jnp.bfloat16)
            s_all = jax.lax.dot_general(
                aq, kr_ref[0:span, :], (((1,), (1,)), ((), ())),
                preferred_element_type=jnp.float32)
            qrb = qrg.astype(jnp.bfloat16)
            ods = []
            for b in range(GQ_):
                qb_ = qrb[b * BLK_:(b + 1) * BLK_, :]
                kb_ = kr_ref[lo + b * BLK_:lo + (b + 1) * BLK_, :DH_]
                ods.append(jax.lax.dot_general(
                    qb_, kb_, (((1,), (1,)), ((), ())),
                    preferred_element_type=jnp.float32) + tri_ref[...])
            s_od = jnp.concatenate(ods, axis=0)
            p_all = jnp.exp(s_all).astype(jnp.bfloat16)
            p_od = jnp.exp(s_od).astype(jnp.bfloat16)
            av_ = jax.lax.dot_general(
                p_all, va_ref[0:span, :], (((1,), (0,)), ((), ())),
                preferred_element_type=jnp.float32)
            oacc = []
            for b in range(GQ_):
                oacc.append(jax.lax.dot_general(
                    p_od[b * BLK_:(b + 1) * BLK_, :],
                    va_ref[lo + b * BLK_:lo + (b + 1) * BLK_, :],
                    (((1,), (0,)), ((), ())),
                    preferred_element_type=jnp.float32))
            av_ = av_ + jnp.concatenate(oacc, axis=0)
            fine_ref[lo:span, g * DH_:(g + 1) * DH_] = (
                av_[:, :DH_] / av_[:, DH_:DH_ + 1])


def _proj_body(comp_ref, fine_ref, g_ref, e0_ref, e1_ref, w_ref, out_ref):
    g = g_ref[...]
    g0 = jnp.dot(g, e0_ref[...], preferred_element_type=jnp.float32)
    g1 = jnp.dot(g, e1_ref[...], preferred_element_type=jnp.float32)
    oh = g0 * comp_ref[...] + g1 * fine_ref[...]
    out_ref[...] = jax.lax.dot_general(
        oh, w_ref[...], (((1,), (1,)), ((), ())),
        preferred_element_type=jnp.float32)


def kernel(inp, norm_w, W_qkv, k_pos, v_pos, mem_kv, Wk1, bk1, Wk2, bk2,
           Wv1, bv1, Wv2, bv2, Wg, bg, Wo):
    x2 = inp.reshape(S_, DIM_)
    aux = jnp.asarray(_AUXN)
    tri = jnp.asarray(_TRIN)

    na = S_ // _RA
    row_spec = lambda w: pl.BlockSpec((_RA, w), lambda i: (i, 0))
    hs_spec = pl.BlockSpec((KVH_, _RA, DH_), lambda i: (0, i, 0))
    q, k, v, g = pl.pallas_call(
        _qkv_body,
        grid=(na,),
        in_specs=[row_spec(DIM_),
                  pl.BlockSpec(((H_ + 2 * KVH_) * DH_, DIM_),
                               lambda i: (0, 0)),
                  pl.BlockSpec((2 * H_, DIM_), lambda i: (0, 0)),
                  pl.BlockSpec((1, DIM_), lambda i: (0, 0)),
                  pl.BlockSpec((1, 2 * H_), lambda i: (0, 0))],
        out_specs=[row_spec(H_ * DH_), hs_spec, hs_spec, row_spec(2 * H_)],
        out_shape=[jax.ShapeDtypeStruct((S_, H_ * DH_), jnp.float32),
                   jax.ShapeDtypeStruct((KVH_, S_, DH_), jnp.float32),
                   jax.ShapeDtypeStruct((KVH_, S_, DH_), jnp.float32),
                   jax.ShapeDtypeStruct((S_, 2 * H_), jnp.float32)],
    )(x2, W_qkv, Wg, norm_w.reshape(1, DIM_), bg.reshape(1, 2 * H_))

    ak = k.reshape(KVH_ * NB_, HID_)
    av = v.reshape(KVH_ * NB_, HID_)
    kp = k_pos.reshape(KVH_, HID_)
    vp = v_pos.reshape(KVH_, HID_)
    nt = HID_ // _HT
    ck_b, cv_b = pl.pallas_call(
        _comp_body,
        grid=(nt,),
        in_specs=[pl.BlockSpec((KVH_ * NB_, HID_), lambda t: (0, 0)),
                  pl.BlockSpec((KVH_ * NB_, HID_), lambda t: (0, 0)),
                  pl.BlockSpec((KVH_, HID_), lambda t: (0, 0)),
                  pl.BlockSpec((KVH_, HID_), lambda t: (0, 0)),
                  pl.BlockSpec((_HT, HID_), lambda t: (t, 0)),
                  pl.BlockSpec((1, _HT), lambda t: (0, t)),
                  pl.BlockSpec((DH_, _HT), lambda t: (0, t)),
                  pl.BlockSpec((_HT, HID_), lambda t: (t, 0)),
                  pl.BlockSpec((1, _HT), lambda t: (0, t)),
                  pl.BlockSpec((DH_, _HT), lambda t: (0, t)),
                  pl.BlockSpec((1, DH_), lambda t: (0, 0)),
                  pl.BlockSpec((1, DH_), lambda t: (0, 0))],
        out_specs=[pl.BlockSpec((KVH_ * NB_, DH_), lambda t: (0, 0)),
                   pl.BlockSpec((KVH_ * NB_, DH_), lambda t: (0, 0))],
        out_shape=[jax.ShapeDtypeStruct((KVH_ * NB_, DH_), jnp.float32),
                   jax.ShapeDtypeStruct((KVH_ * NB_, DH_), jnp.float32)],
    )(ak, av, kp, vp, Wk1, bk1.reshape(1, HID_), Wk2,
      Wv1, bv1.reshape(1, HID_), Wv2,
      bk2.reshape(1, DH_), bv2.reshape(1, DH_))

    ck = jnp.concatenate([mem_kv[0], ck_b.reshape(KVH_, NB_, DH_)], axis=1)
    cv = jnp.concatenate([mem_kv[1], cv_b.reshape(KVH_, NB_, DH_)], axis=1)

    comp_n, fine_n = pl.pallas_call(
        _attn_body,
        grid=(KVH_,),
        in_specs=[
            pl.BlockSpec((S_, GQ_ * DH_), lambda h: (0, h)),
            pl.BlockSpec((1, S_, DH_), lambda h: (h, 0, 0)),
            pl.BlockSpec((1, S_, DH_), lambda h: (h, 0, 0)),
            pl.BlockSpec((1, NB_ + NMEM_, DH_), lambda h: (h, 0, 0)),
            pl.BlockSpec((1, NB_ + NMEM_, DH_), lambda h: (h, 0, 0)),
            pl.BlockSpec((S_, 4 * DH_), lambda h: (0, 0)),
            pl.BlockSpec((BLK_, BLK_), lambda h: (0, 0)),
        ],
        out_specs=[pl.BlockSpec((S_, GQ_ * DH_), lambda h: (0, h)),
                   pl.BlockSpec((S_, GQ_ * DH_), lambda h: (0, h))],
        out_shape=[jax.ShapeDtypeStruct((S_, H_ * DH_), jnp.float32),
                   jax.ShapeDtypeStruct((S_, H_ * DH_), jnp.float32)],
        scratch_shapes=[pltpu.VMEM((S_, _AUG), jnp.bfloat16),
                        pltpu.VMEM((S_, DH_ + 1), jnp.bfloat16)],
    )(q, k, v, ck, cv, aux, tri)

    e0 = jnp.asarray(_E0N)
    e1 = jnp.asarray(_E1N)

    out = pl.pallas_call(
        _proj_body,
        grid=(na,),
        in_specs=[row_spec(H_ * DH_), row_spec(H_ * DH_), row_spec(2 * H_),
                  pl.BlockSpec((2 * H_, H_ * DH_), lambda i: (0, 0)),
                  pl.BlockSpec((2 * H_, H_ * DH_), lambda i: (0, 0)),
                  pl.BlockSpec((H_ * DH_, DIM_), lambda i: (0, 0))],
        out_specs=row_spec(DIM_),
        out_shape=jax.ShapeDtypeStruct((S_, DIM_), jnp.float32),
    )(comp_n, fine_n, g, e0, e1, Wo)
    return out.reshape(1, S_, DIM_)


# speedup vs baseline: 7.0811x; 1.1303x over previous
import jax
import jax.numpy as jnp
import numpy as np
from jax.experimental import pallas as pl
from jax.experimental.pallas import tpu as pltpu

S_ = 2048
DIM_ = 1024
DH_ = 64
H_ = 16
KVH_ = 4
GQ_ = H_ // KVH_
BLK_ = 64
NB_ = S_ // BLK_
NSEL_ = 8
NMEM_ = 1
HID_ = BLK_ * DH_
SCALE_ = DH_ ** -0.5
MNEG_ = float(-jnp.finfo(jnp.float32).max)
EPS_ = float(jnp.finfo(jnp.float32).eps)

_RA = 256
_HT = 512
_GP = 256
_NG = S_ // _GP
_AUG = DH_ + NB_

_posn = np.arange(S_, dtype=np.float64)
_invn = 1.0 / (10000.0 ** (np.arange(0, DH_, 2, dtype=np.float64) / DH_))
_angn = _posn[:, None] * _invn[None, :]
_cosn = np.repeat(np.cos(_angn), 2, axis=-1).astype(np.float32)
_sinn = np.repeat(np.sin(_angn), 2, axis=-1).astype(np.float32)
_mtn = np.repeat(np.eye(NB_, dtype=np.float32), BLK_, axis=1).T
_AUXN = np.concatenate(
    [_cosn, _sinn, _mtn, np.zeros((S_, 2 * DH_ - NB_), np.float32)], axis=1)
_tin = np.arange(BLK_)
_TRIN = np.where(_tin[None, :] <= _tin[:, None], 0.0, -1e30).astype(np.float32)
_hhn = np.arange(H_ * DH_) // DH_
_E0N = np.zeros((2 * H_, H_ * DH_), np.float32)
_E0N[2 * _hhn, np.arange(H_ * DH_)] = 1.0
_E1N = np.zeros((2 * H_, H_ * DH_), np.float32)
_E1N[2 * _hhn + 1, np.arange(H_ * DH_)] = 1.0


def _rot(t, cos, sin):
    a = jnp.concatenate([t[:, -1:], t[:, :-1]], axis=1)
    b = jnp.concatenate([t[:, 1:], t[:, :1]], axis=1)
    lane = jax.lax.broadcasted_iota(jnp.int32, t.shape, 1)
    rh = jnp.where(lane % 2 == 0, -b, a)
    return t * cos + rh * sin


def _qkv_body(x_ref, wt_ref, wg_ref, nw_ref, bg_ref, q_ref, k_ref, v_ref,
              g_ref):
    x = x_ref[...]
    ms = jnp.mean(x * x, axis=1, keepdims=True)
    x = x * jax.lax.rsqrt(ms + EPS_) * nw_ref[...]
    y = jax.lax.dot_general(x, wt_ref[...], (((1,), (1,)), ((), ())),
                            preferred_element_type=jnp.float32)
    gr = jax.lax.dot_general(x, wg_ref[...], (((1,), (1,)), ((), ())),
                             preferred_element_type=jnp.float32)
    q_ref[...] = y[:, :H_ * DH_]
    for h in range(KVH_):
        k_ref[h, :, :] = y[:, (H_ + h) * DH_:(H_ + h + 1) * DH_]
        v_ref[h, :, :] = y[:, (H_ + KVH_ + h) * DH_:(H_ + KVH_ + h + 1) * DH_]
    g_ref[...] = jax.nn.sigmoid(gr + bg_ref[...])


def _comp_body(ak_ref, av_ref, kp_ref, vp_ref, w1k_ref, b1k_ref, w2k_ref,
               w1v_ref, b1v_ref, w2v_ref, bk2_ref, bv2_ref, ck_ref, cv_ref):
    t = pl.program_id(0)

    @pl.when(t == 0)
    def _():
        ck_ref[...] = jnp.broadcast_to(bk2_ref[...], ck_ref.shape)
        cv_ref[...] = jnp.broadcast_to(bv2_ref[...], cv_ref.shape)

    ak = (ak_ref[...].reshape(KVH_, NB_, HID_)
          + kp_ref[...][:, None, :]).reshape(KVH_ * NB_, HID_)
    av = (av_ref[...].reshape(KVH_, NB_, HID_)
          + vp_ref[...][:, None, :]).reshape(KVH_ * NB_, HID_)
    hk = jnp.maximum(
        jax.lax.dot_general(ak, w1k_ref[...], (((1,), (1,)), ((), ())),
                            preferred_element_type=jnp.float32)
        + b1k_ref[...], 0.0)
    hv = jnp.maximum(
        jax.lax.dot_general(av, w1v_ref[...], (((1,), (1,)), ((), ())),
                            preferred_element_type=jnp.float32)
        + b1v_ref[...], 0.0)
    ck_ref[...] += jax.lax.dot_general(
        hk, w2k_ref[...], (((1,), (1,)), ((), ())),
        preferred_element_type=jnp.float32)
    cv_ref[...] += jax.lax.dot_general(
        hv, w2v_ref[...], (((1,), (1,)), ((), ())),
        preferred_element_type=jnp.float32)


def _attn_body(q_ref, k_ref, v_ref, ck_ref, cv_ref, aux_ref, tri_ref,
               comp_ref, fine_ref, kr_ref, va_ref):
    kr_ref[...] = jnp.concatenate(
        [_rot(k_ref[0], aux_ref[:, :DH_], aux_ref[:, DH_:2 * DH_]),
         aux_ref[:, 2 * DH_:2 * DH_ + NB_]], axis=1).astype(jnp.bfloat16)
    va_ref[...] = jnp.concatenate(
        [v_ref[0], jnp.ones((S_, 1), jnp.float32)], axis=1).astype(jnp.bfloat16)
    ckt = ck_ref[0]
    cva = jnp.concatenate(
        [cv_ref[0], jnp.ones((NB_ + NMEM_, 1), jnp.float32)], axis=1)
    for qg in range(_NG):
        lo = qg * _GP
        span = lo + _GP
        cosg = aux_ref[lo:span, :DH_]
        sing = aux_ref[lo:span, DH_:2 * DH_]
        qpgs = []
        csims = []
        for g in range(GQ_):
            qpg = q_ref[lo:span, g * DH_:(g + 1) * DH_]
            qpgs.append(qpg)
            csims.append(jax.lax.dot_general(
                qpg, ckt, (((1,), (1,)), ((), ())),
                preferred_element_type=jnp.float32) * SCALE_)
        ri = jax.lax.broadcasted_iota(jnp.int32, csims[0].shape, 0)
        ci = jax.lax.broadcasted_iota(jnp.int32, csims[0].shape, 1)
        cvis = ci < (lo + ri) // BLK_ + 1
        imp = jnp.zeros(csims[0].shape, jnp.float32)
        for g in range(GQ_):
            cs = jnp.where(cvis, csims[g], MNEG_)
            imp = imp + cs
            cp = jnp.exp(cs)
            cav = jnp.dot(cp, cva, preferred_element_type=jnp.float32)
            comp_ref[lo:span, g * DH_:(g + 1) * DH_] = (
                cav[:, :DH_] / cav[:, DH_:DH_ + 1])

        imp = imp[:, NMEM_:] * (1.0 / GQ_)
        m2 = jnp.maximum(jnp.max(imp, axis=1, keepdims=True), -1e3)
        e = jnp.exp(imp - m2)
        probs = e / (jnp.sum(e, axis=1, keepdims=True) + jnp.exp(-1e3 - m2))

        work = probs.T
        rowi = jax.lax.broadcasted_iota(jnp.int32, work.shape, 0)
        sel_t = jnp.full(work.shape, -1e30, jnp.float32)
        for _ in range(NSEL_):
            m = jnp.max(work, axis=0, keepdims=True)
            cand = work == m
            idxv = jnp.where(cand, rowi, NB_ * 2)
            pick = rowi == jnp.min(idxv, axis=0, keepdims=True)
            sel_t = jnp.where(pick & (m > 1e-10), 0.0, sel_t)
            work = jnp.where(pick, -1.0, work)
        sel_pos = sel_t.T

        for g in range(GQ_):
            qrg = _rot(qpgs[g], cosg, sing) * SCALE_
            aq = jnp.concatenate(
                [qrg, sel_pos], axis=1).astype(jnp.bfloat16)
            s_all = jax.lax.dot_general(
                aq, kr_ref[0:span, :], (((1,), (1,)), ((), ())),
                preferred_element_type=jnp.float32)
            qrb = qrg.astype(jnp.bfloat16)
            ods = []
            for b in range(GQ_):
                qb_ = qrb[b * BLK_:(b + 1) * BLK_, :]
                kb_ = kr_ref[lo + b * BLK_:lo + (b + 1) * BLK_, :DH_]
                ods.append(jax.lax.dot_general(
                    qb_, kb_, (((1,), (1,)), ((), ())),
                    preferred_element_type=jnp.float32) + tri_ref[...])
            s_od = jnp.concatenate(ods, axis=0)
            p_all = jnp.exp(s_all).astype(jnp.bfloat16)
            p_od = jnp.exp(s_od).astype(jnp.bfloat16)
            av_ = jax.lax.dot_general(
                p_all, va_ref[0:span, :], (((1,), (0,)), ((), ())),
                preferred_element_type=jnp.float32)
            oacc = []
            for b in range(GQ_):
                oacc.append(jax.lax.dot_general(
                    p_od[b * BLK_:(b + 1) * BLK_, :],
                    va_ref[lo + b * BLK_:lo + (b + 1) * BLK_, :],
                    (((1,), (0,)), ((), ())),
                    preferred_element_type=jnp.float32))
            av_ = av_ + jnp.concatenate(oacc, axis=0)
            fine_ref[lo:span, g * DH_:(g + 1) * DH_] = (
                av_[:, :DH_] / av_[:, DH_:DH_ + 1])


def _proj_body(comp_ref, fine_ref, g_ref, e0_ref, e1_ref, w_ref, out_ref):
    g = g_ref[...]
    g0 = jnp.dot(g, e0_ref[...], preferred_element_type=jnp.float32)
    g1 = jnp.dot(g, e1_ref[...], preferred_element_type=jnp.float32)
    oh = g0 * comp_ref[...] + g1 * fine_ref[...]
    out_ref[...] = jax.lax.dot_general(
        oh, w_ref[...], (((1,), (1,)), ((), ())),
        preferred_element_type=jnp.float32)


def kernel(inp, norm_w, W_qkv, k_pos, v_pos, mem_kv, Wk1, bk1, Wk2, bk2,
           Wv1, bv1, Wv2, bv2, Wg, bg, Wo):
    x2 = inp.reshape(S_, DIM_)
    aux = jnp.asarray(_AUXN)
    tri = jnp.asarray(_TRIN)

    na = S_ // _RA
    row_spec = lambda w: pl.BlockSpec((_RA, w), lambda i: (i, 0))
    hs_spec = pl.BlockSpec((KVH_, _RA, DH_), lambda i: (0, i, 0))
    q, k, v, g = pl.pallas_call(
        _qkv_body,
        grid=(na,),
        in_specs=[row_spec(DIM_),
                  pl.BlockSpec(((H_ + 2 * KVH_) * DH_, DIM_),
                               lambda i: (0, 0)),
                  pl.BlockSpec((2 * H_, DIM_), lambda i: (0, 0)),
                  pl.BlockSpec((1, DIM_), lambda i: (0, 0)),
                  pl.BlockSpec((1, 2 * H_), lambda i: (0, 0))],
        out_specs=[row_spec(H_ * DH_), hs_spec, hs_spec, row_spec(2 * H_)],
        out_shape=[jax.ShapeDtypeStruct((S_, H_ * DH_), jnp.float32),
                   jax.ShapeDtypeStruct((KVH_, S_, DH_), jnp.float32),
                   jax.ShapeDtypeStruct((KVH_, S_, DH_), jnp.float32),
                   jax.ShapeDtypeStruct((S_, 2 * H_), jnp.float32)],
    )(x2, W_qkv, Wg, norm_w.reshape(1, DIM_), bg.reshape(1, 2 * H_))

    ak = k.reshape(KVH_ * NB_, HID_)
    av = v.reshape(KVH_ * NB_, HID_)
    kp = k_pos.reshape(KVH_, HID_)
    vp = v_pos.reshape(KVH_, HID_)
    nt = HID_ // _HT
    ck_b, cv_b = pl.pallas_call(
        _comp_body,
        grid=(nt,),
        in_specs=[pl.BlockSpec((KVH_ * NB_, HID_), lambda t: (0, 0)),
                  pl.BlockSpec((KVH_ * NB_, HID_), lambda t: (0, 0)),
                  pl.BlockSpec((KVH_, HID_), lambda t: (0, 0)),
                  pl.BlockSpec((KVH_, HID_), lambda t: (0, 0)),
                  pl.BlockSpec((_HT, HID_), lambda t: (t, 0)),
                  pl.BlockSpec((1, _HT), lambda t: (0, t)),
                  pl.BlockSpec((DH_, _HT), lambda t: (0, t)),
                  pl.BlockSpec((_HT, HID_), lambda t: (t, 0)),
                  pl.BlockSpec((1, _HT), lambda t: (0, t)),
                  pl.BlockSpec((DH_, _HT), lambda t: (0, t)),
                  pl.BlockSpec((1, DH_), lambda t: (0, 0)),
                  pl.BlockSpec((1, DH_), lambda t: (0, 0))],
        out_specs=[pl.BlockSpec((KVH_ * NB_, DH_), lambda t: (0, 0)),
                   pl.BlockSpec((KVH_ * NB_, DH_), lambda t: (0, 0))],
        out_shape=[jax.ShapeDtypeStruct((KVH_ * NB_, DH_), jnp.float32),
                   jax.ShapeDtypeStruct((KVH_ * NB_, DH_), jnp.float32)],
    )(ak, av, kp, vp, Wk1, bk1.reshape(1, HID_), Wk2,
      Wv1, bv1.reshape(1, HID_), Wv2,
      bk2.reshape(1, DH_), bv2.reshape(1, DH_))

    ck = jnp.concatenate([mem_kv[0], ck_b.reshape(KVH_, NB_, DH_)], axis=1)
    cv = jnp.concatenate([mem_kv[1], cv_b.reshape(KVH_, NB_, DH_)], axis=1)

    comp_n, fine_n = pl.pallas_call(
        _attn_body,
        grid=(KVH_,),
        in_specs=[
            pl.BlockSpec((S_, GQ_ * DH_), lambda h: (0, h)),
            pl.BlockSpec((1, S_, DH_), lambda h: (h, 0, 0)),
            pl.BlockSpec((1, S_, DH_), lambda h: (h, 0, 0)),
            pl.BlockSpec((1, NB_ + NMEM_, DH_), lambda h: (h, 0, 0)),
            pl.BlockSpec((1, NB_ + NMEM_, DH_), lambda h: (h, 0, 0)),
            pl.BlockSpec((S_, 4 * DH_), lambda h: (0, 0)),
            pl.BlockSpec((BLK_, BLK_), lambda h: (0, 0)),
        ],
        out_specs=[pl.BlockSpec((S_, GQ_ * DH_), lambda h: (0, h)),
                   pl.BlockSpec((S_, GQ_ * DH_), lambda h: (0, h))],
        out_shape=[jax.ShapeDtypeStruct((S_, H_ * DH_), jnp.float32),
                   jax.ShapeDtypeStruct((S_, H_ * DH_), jnp.float32)],
        scratch_shapes=[pltpu.VMEM((S_, _AUG), jnp.bfloat16),
                        pltpu.VMEM((S_, DH_ + 1), jnp.bfloat16)],
    )(q, k, v, ck, cv, aux, tri)

    e0 = jnp.asarray(_E0N)
    e1 = jnp.asarray(_E1N)

    out = pl.pallas_call(
        _proj_body,
        grid=(na,),
        in_specs=[row_spec(H_ * DH_), row_spec(H_ * DH_), row_spec(2 * H_),
                  pl.BlockSpec((2 * H_, H_ * DH_), lambda i: (0, 0)),
                  pl.BlockSpec((2 * H_, H_ * DH_), lambda i: (0, 0)),
                  pl.BlockSpec((H_ * DH_, DIM_), lambda i: (0, 0))],
        out_specs=row_spec(DIM_),
        out_shape=jax.ShapeDtypeStruct((S_, DIM_), jnp.float32),
    )(comp_n, fine_n, g, e0, e1, Wo)
    return out.reshape(1, S_, DIM_)
